# initial kernel scaffold (unmeasured)
import jax
import jax.numpy as jnp
from jax import lax
from jax.experimental import pallas as pl
from jax.experimental.pallas import tpu as pltpu

NY = 4
E_LOCAL = 2
T = 1024
D = 1024
F = 2048
RT = 512


def kernel(x, assign, W1, W2):
    xb = x.astype(jnp.bfloat16)
    w1b = W1.astype(jnp.bfloat16)
    w2b = W2.astype(jnp.bfloat16)
    a2 = assign.reshape(T, 1)

    def body(x_ref, a_ref, w1_ref, w2_ref, out_ref,
             X_all, A_all, acc, orecv,
             x_send, x_recv, a_send, a_recv, o_send, o_recv):
        my_x = lax.axis_index("x")
        my_y = lax.axis_index("y")
        my_z = lax.axis_index("z")

        bsem = pltpu.get_barrier_semaphore()
        for dy in range(1, NY):
            peer = (my_y + dy) % NY
            pl.semaphore_signal(bsem, inc=1, device_id=(my_x, peer, my_z),
                                device_id_type=pl.DeviceIdType.MESH)
        pl.semaphore_wait(bsem, NY - 1)

        X_all[my_y] = x_ref[...]
        A_all[my_y] = a_ref[...]

        in_rdmas = []
        for dy in range(1, NY):
            p = (my_y + dy) % NY
            r = pltpu.make_async_remote_copy(
                src_ref=X_all.at[my_y], dst_ref=X_all.at[my_y],
                send_sem=x_send.at[p], recv_sem=x_recv.at[my_y],
                device_id=(my_x, p, my_z),
                device_id_type=pl.DeviceIdType.MESH)
            r.start()
            in_rdmas.append(r)
            r = pltpu.make_async_remote_copy(
                src_ref=A_all.at[my_y], dst_ref=A_all.at[my_y],
                send_sem=a_send.at[p], recv_sem=a_recv.at[my_y],
                device_id=(my_x, p, my_z),
                device_id_type=pl.DeviceIdType.MESH)
            r.start()
            in_rdmas.append(r)

        for db in range(NY):
            b = (my_y + db) % NY
            if db > 0:
                pltpu.make_async_remote_copy(
                    src_ref=X_all.at[b], dst_ref=X_all.at[b],
                    send_sem=x_send.at[b], recv_sem=x_recv.at[b],
                    device_id=(my_x, b, my_z),
                    device_id_type=pl.DeviceIdType.MESH).wait_recv()
                pltpu.make_async_remote_copy(
                    src_ref=A_all.at[b], dst_ref=A_all.at[b],
                    send_sem=a_send.at[b], recv_sem=a_recv.at[b],
                    device_id=(my_x, b, my_z),
                    device_id_type=pl.DeviceIdType.MESH).wait_recv()
            for r0 in range(0, T, RT):
                xt = X_all[b, pl.ds(r0, RT), :]
                at = A_all[b, pl.ds(r0, RT), :]
                tot = None
                for le in range(E_LOCAL):
                    e = my_y * E_LOCAL + le
                    h = jnp.maximum(
                        jnp.dot(xt, w1_ref[le],
                                preferred_element_type=jnp.float32), 0.0)
                    o = jnp.dot(h.astype(jnp.bfloat16), w2_ref[le],
                                preferred_element_type=jnp.float32)
                    om = jnp.where(at == e, o, 0.0)
                    tot = om if tot is None else tot + om
                acc[b, pl.ds(r0, RT), :] = tot.astype(jnp.bfloat16)

        for r in in_rdmas:
            r.wait_send()

        o_rdmas = []
        for dy in range(1, NY):
            p = (my_y + dy) % NY
            r = pltpu.make_async_remote_copy(
                src_ref=acc.at[p], dst_ref=orecv.at[my_y],
                send_sem=o_send.at[p], recv_sem=o_recv.at[my_y],
                device_id=(my_x, p, my_z),
                device_id_type=pl.DeviceIdType.MESH)
            r.start()
            o_rdmas.append(r)

        orecv[my_y] = acc[my_y]

        for dy in range(1, NY):
            s = (my_y + dy) % NY
            pltpu.make_async_remote_copy(
                src_ref=acc.at[s], dst_ref=orecv.at[s],
                send_sem=o_send.at[s], recv_sem=o_recv.at[s],
                device_id=(my_x, s, my_z),
                device_id_type=pl.DeviceIdType.MESH).wait_recv()
        for r in o_rdmas:
            r.wait_send()

        out_ref[...] = (orecv[0].astype(jnp.float32)
                        + orecv[1].astype(jnp.float32)
                        + orecv[2].astype(jnp.float32)
                        + orecv[3].astype(jnp.float32))

    return pl.pallas_call(
        body,
        out_shape=jax.ShapeDtypeStruct((T, D), jnp.float32),
        in_specs=[pl.BlockSpec(memory_space=pltpu.VMEM)] * 4,
        out_specs=pl.BlockSpec(memory_space=pltpu.VMEM),
        scratch_shapes=[
            pltpu.VMEM((NY, T, D), jnp.bfloat16),
            pltpu.VMEM((NY, T, 1), jnp.int32),
            pltpu.VMEM((NY, T, D), jnp.bfloat16),
            pltpu.VMEM((NY, T, D), jnp.bfloat16),
            pltpu.SemaphoreType.DMA((NY,)),
            pltpu.SemaphoreType.DMA((NY,)),
            pltpu.SemaphoreType.DMA((NY,)),
            pltpu.SemaphoreType.DMA((NY,)),
            pltpu.SemaphoreType.DMA((NY,)),
            pltpu.SemaphoreType.DMA((NY,)),
        ],
        compiler_params=pltpu.CompilerParams(collective_id=0),
    )(xb, a2, w1b, w2b)


# baseline (device time: 276992 ns/iter reference)
import jax
import jax.numpy as jnp
from jax import lax
from jax.experimental import pallas as pl
from jax.experimental.pallas import tpu as pltpu

NY = 4
E_LOCAL = 2
T = 1024
D = 1024
F = 2048
RT = 256


def kernel(x, assign, W1, W2):
    xb = x.astype(jnp.bfloat16)
    w1b = W1.astype(jnp.bfloat16)
    w2b = W2.astype(jnp.bfloat16)
    a2 = assign.reshape(T, 1)

    def body(x_ref, a_ref, w1_ref, w2_ref, out_ref,
             X_all, A_all, acc, orecv,
             x_send, x_recv, a_send, a_recv, o_send, o_recv):
        my_x = lax.axis_index("x")
        my_y = lax.axis_index("y")
        my_z = lax.axis_index("z")

        bsem = pltpu.get_barrier_semaphore()
        for dy in range(1, NY):
            peer = (my_y + dy) % NY
            pl.semaphore_signal(bsem, inc=1, device_id=(my_x, peer, my_z),
                                device_id_type=pl.DeviceIdType.MESH)
        pl.semaphore_wait(bsem, NY - 1)

        X_all[my_y] = x_ref[...]
        A_all[my_y] = a_ref[...]

        in_rdmas = []
        for dy in range(1, NY):
            p = (my_y + dy) % NY
            r = pltpu.make_async_remote_copy(
                src_ref=X_all.at[my_y], dst_ref=X_all.at[my_y],
                send_sem=x_send.at[p], recv_sem=x_recv.at[my_y],
                device_id=(my_x, p, my_z),
                device_id_type=pl.DeviceIdType.MESH)
            r.start()
            in_rdmas.append(r)
            r = pltpu.make_async_remote_copy(
                src_ref=A_all.at[my_y], dst_ref=A_all.at[my_y],
                send_sem=a_send.at[p], recv_sem=a_recv.at[my_y],
                device_id=(my_x, p, my_z),
                device_id_type=pl.DeviceIdType.MESH)
            r.start()
            in_rdmas.append(r)

        for db in range(NY):
            b = (my_y + db) % NY
            if db > 0:
                pltpu.make_async_remote_copy(
                    src_ref=X_all.at[b], dst_ref=X_all.at[b],
                    send_sem=x_send.at[b], recv_sem=x_recv.at[b],
                    device_id=(my_x, b, my_z),
                    device_id_type=pl.DeviceIdType.MESH).wait_recv()
                pltpu.make_async_remote_copy(
                    src_ref=A_all.at[b], dst_ref=A_all.at[b],
                    send_sem=a_send.at[b], recv_sem=a_recv.at[b],
                    device_id=(my_x, b, my_z),
                    device_id_type=pl.DeviceIdType.MESH).wait_recv()
            def tile_body(i, _):
                r0 = i * RT
                xt = X_all[b, pl.ds(r0, RT), :]
                at = A_all[b, pl.ds(r0, RT), :]
                tot = None
                for le in range(E_LOCAL):
                    e = my_y * E_LOCAL + le
                    h = jnp.maximum(
                        jnp.dot(xt, w1_ref[le],
                                preferred_element_type=jnp.float32), 0.0)
                    o = jnp.dot(h.astype(jnp.bfloat16), w2_ref[le],
                                preferred_element_type=jnp.float32)
                    om = jnp.where(at == e, o, 0.0)
                    tot = om if tot is None else tot + om
                acc[b, pl.ds(r0, RT), :] = tot.astype(jnp.bfloat16)
                return 0

            lax.fori_loop(0, T // RT, tile_body, 0)

        for r in in_rdmas:
            r.wait_send()

        o_rdmas = []
        for dy in range(1, NY):
            p = (my_y + dy) % NY
            r = pltpu.make_async_remote_copy(
                src_ref=acc.at[p], dst_ref=orecv.at[my_y],
                send_sem=o_send.at[p], recv_sem=o_recv.at[my_y],
                device_id=(my_x, p, my_z),
                device_id_type=pl.DeviceIdType.MESH)
            r.start()
            o_rdmas.append(r)

        orecv[my_y] = acc[my_y]

        for dy in range(1, NY):
            s = (my_y + dy) % NY
            pltpu.make_async_remote_copy(
                src_ref=acc.at[s], dst_ref=orecv.at[s],
                send_sem=o_send.at[s], recv_sem=o_recv.at[s],
                device_id=(my_x, s, my_z),
                device_id_type=pl.DeviceIdType.MESH).wait_recv()
        for r in o_rdmas:
            r.wait_send()

        out_ref[...] = (orecv[0].astype(jnp.float32)
                        + orecv[1].astype(jnp.float32)
                        + orecv[2].astype(jnp.float32)
                        + orecv[3].astype(jnp.float32))

    return pl.pallas_call(
        body,
        out_shape=jax.ShapeDtypeStruct((T, D), jnp.float32),
        in_specs=[pl.BlockSpec(memory_space=pltpu.VMEM)] * 4,
        out_specs=pl.BlockSpec(memory_space=pltpu.VMEM),
        scratch_shapes=[
            pltpu.VMEM((NY, T, D), jnp.bfloat16),
            pltpu.VMEM((NY, T, 1), jnp.int32),
            pltpu.VMEM((NY, T, D), jnp.bfloat16),
            pltpu.VMEM((NY, T, D), jnp.bfloat16),
            pltpu.SemaphoreType.DMA((NY,)),
            pltpu.SemaphoreType.DMA((NY,)),
            pltpu.SemaphoreType.DMA((NY,)),
            pltpu.SemaphoreType.DMA((NY,)),
            pltpu.SemaphoreType.DMA((NY,)),
            pltpu.SemaphoreType.DMA((NY,)),
        ],
        compiler_params=pltpu.CompilerParams(
            collective_id=0, vmem_limit_bytes=56 * 1024 * 1024),
    )(xb, a2, w1b, w2b)


# device time: 258582 ns/iter; 1.0712x vs baseline; 1.0712x over previous
import jax
import jax.numpy as jnp
from jax import lax
from jax.experimental import pallas as pl
from jax.experimental.pallas import tpu as pltpu

NY = 4
E_LOCAL = 2
T = 1024
D = 1024
F = 2048
RT = 256


def kernel(x, assign, W1, W2):
    xb = x.astype(jnp.bfloat16)
    w1b = W1.astype(jnp.bfloat16)
    w2b = W2.astype(jnp.bfloat16)
    a2 = assign.reshape(T, 1)

    def body(x_ref, a_ref, w1_ref, w2_ref, out_ref,
             X_all, A_all, acc, orecv,
             x_send, x_recv, a_send, a_recv, o_send, o_recv):
        my_x = lax.axis_index("x")
        my_y = lax.axis_index("y")
        my_z = lax.axis_index("z")

        bsem = pltpu.get_barrier_semaphore()
        for dy in range(1, NY):
            peer = (my_y + dy) % NY
            pl.semaphore_signal(bsem, inc=1, device_id=(my_x, peer, my_z),
                                device_id_type=pl.DeviceIdType.MESH)
        pl.semaphore_wait(bsem, NY - 1)

        X_all[my_y] = x_ref[...]
        A_all[my_y] = a_ref[...]

        in_rdmas = []
        for dy in range(1, NY):
            p = (my_y + dy) % NY
            r = pltpu.make_async_remote_copy(
                src_ref=X_all.at[my_y], dst_ref=X_all.at[my_y],
                send_sem=x_send.at[p], recv_sem=x_recv.at[my_y],
                device_id=(my_x, p, my_z),
                device_id_type=pl.DeviceIdType.MESH)
            r.start()
            in_rdmas.append(r)
            r = pltpu.make_async_remote_copy(
                src_ref=A_all.at[my_y], dst_ref=A_all.at[my_y],
                send_sem=a_send.at[p], recv_sem=a_recv.at[my_y],
                device_id=(my_x, p, my_z),
                device_id_type=pl.DeviceIdType.MESH)
            r.start()
            in_rdmas.append(r)

        o_rdmas = []
        for db in range(NY):
            b = (my_y + db) % NY
            if db > 0:
                pltpu.make_async_remote_copy(
                    src_ref=X_all.at[b], dst_ref=X_all.at[b],
                    send_sem=x_send.at[b], recv_sem=x_recv.at[b],
                    device_id=(my_x, b, my_z),
                    device_id_type=pl.DeviceIdType.MESH).wait_recv()
                pltpu.make_async_remote_copy(
                    src_ref=A_all.at[b], dst_ref=A_all.at[b],
                    send_sem=a_send.at[b], recv_sem=a_recv.at[b],
                    device_id=(my_x, b, my_z),
                    device_id_type=pl.DeviceIdType.MESH).wait_recv()
            def tile_body(i, _):
                r0 = i * RT
                xt = X_all[b, pl.ds(r0, RT), :]
                at = A_all[b, pl.ds(r0, RT), :]
                tot = None
                for le in range(E_LOCAL):
                    e = my_y * E_LOCAL + le
                    h = jnp.maximum(
                        jnp.dot(xt, w1_ref[le],
                                preferred_element_type=jnp.float32), 0.0)
                    o = jnp.dot(h.astype(jnp.bfloat16), w2_ref[le],
                                preferred_element_type=jnp.float32)
                    om = jnp.where(at == e, o, 0.0)
                    tot = om if tot is None else tot + om
                acc[b, pl.ds(r0, RT), :] = tot.astype(jnp.bfloat16)
                return 0

            lax.fori_loop(0, T // RT, tile_body, 0)

            if db == 0:
                orecv[my_y] = acc[my_y]
            else:
                r = pltpu.make_async_remote_copy(
                    src_ref=acc.at[b], dst_ref=orecv.at[my_y],
                    send_sem=o_send.at[b], recv_sem=o_recv.at[my_y],
                    device_id=(my_x, b, my_z),
                    device_id_type=pl.DeviceIdType.MESH)
                r.start()
                o_rdmas.append(r)

        for r in in_rdmas:
            r.wait_send()

        for dy in range(1, NY):
            s = (my_y + dy) % NY
            pltpu.make_async_remote_copy(
                src_ref=acc.at[s], dst_ref=orecv.at[s],
                send_sem=o_send.at[s], recv_sem=o_recv.at[s],
                device_id=(my_x, s, my_z),
                device_id_type=pl.DeviceIdType.MESH).wait_recv()
        for r in o_rdmas:
            r.wait_send()

        out_ref[...] = (orecv[0].astype(jnp.float32)
                        + orecv[1].astype(jnp.float32)
                        + orecv[2].astype(jnp.float32)
                        + orecv[3].astype(jnp.float32))

    return pl.pallas_call(
        body,
        out_shape=jax.ShapeDtypeStruct((T, D), jnp.float32),
        in_specs=[pl.BlockSpec(memory_space=pltpu.VMEM)] * 4,
        out_specs=pl.BlockSpec(memory_space=pltpu.VMEM),
        scratch_shapes=[
            pltpu.VMEM((NY, T, D), jnp.bfloat16),
            pltpu.VMEM((NY, T, 1), jnp.int32),
            pltpu.VMEM((NY, T, D), jnp.bfloat16),
            pltpu.VMEM((NY, T, D), jnp.bfloat16),
            pltpu.SemaphoreType.DMA((NY,)),
            pltpu.SemaphoreType.DMA((NY,)),
            pltpu.SemaphoreType.DMA((NY,)),
            pltpu.SemaphoreType.DMA((NY,)),
            pltpu.SemaphoreType.DMA((NY,)),
            pltpu.SemaphoreType.DMA((NY,)),
        ],
        compiler_params=pltpu.CompilerParams(
            collective_id=0, vmem_limit_bytes=56 * 1024 * 1024),
    )(xb, a2, w1b, w2b)


# device time: 114194 ns/iter; 2.4256x vs baseline; 2.2644x over previous
import jax
import jax.numpy as jnp
from jax import lax
from jax.experimental import pallas as pl
from jax.experimental.pallas import tpu as pltpu

NY = 4
NZ = 4
E_LOCAL = 2
T = 1024
D = 1024
F = 2048
QT = T // NZ


def kernel(x, assign, W1, W2):
    xb = x.astype(jnp.bfloat16)
    w1b = W1.astype(jnp.bfloat16)
    w2b = W2.astype(jnp.bfloat16)
    a2 = assign.reshape(T, 1)

    def body(x_ref, a_ref, w1_ref, w2_ref, out_ref,
             X_all, A_all, acc, orecv, zbuf,
             x_send, x_recv, a_send, a_recv, o_send, o_recv,
             z_send, z_recv):
        my_x = lax.axis_index("x")
        my_y = lax.axis_index("y")
        my_z = lax.axis_index("z")

        bsem = pltpu.get_barrier_semaphore()
        for dy in range(1, NY):
            pl.semaphore_signal(bsem, inc=1,
                                device_id=(my_x, (my_y + dy) % NY, my_z),
                                device_id_type=pl.DeviceIdType.MESH)
        for dz in range(1, NZ):
            pl.semaphore_signal(bsem, inc=1,
                                device_id=(my_x, my_y, (my_z + dz) % NZ),
                                device_id_type=pl.DeviceIdType.MESH)
        pl.semaphore_wait(bsem, (NY - 1) + (NZ - 1))

        X_all[my_y] = x_ref[pl.ds(my_z * QT, QT), :]
        A_all[my_y] = a_ref[pl.ds(my_z * QT, QT), :]

        in_rdmas = []
        for dy in range(1, NY):
            p = (my_y + dy) % NY
            r = pltpu.make_async_remote_copy(
                src_ref=X_all.at[my_y], dst_ref=X_all.at[my_y],
                send_sem=x_send.at[p], recv_sem=x_recv.at[my_y],
                device_id=(my_x, p, my_z),
                device_id_type=pl.DeviceIdType.MESH)
            r.start()
            in_rdmas.append(r)
            r = pltpu.make_async_remote_copy(
                src_ref=A_all.at[my_y], dst_ref=A_all.at[my_y],
                send_sem=a_send.at[p], recv_sem=a_recv.at[my_y],
                device_id=(my_x, p, my_z),
                device_id_type=pl.DeviceIdType.MESH)
            r.start()
            in_rdmas.append(r)

        o_rdmas = []
        for db in range(NY):
            b = (my_y + db) % NY
            if db > 0:
                pltpu.make_async_remote_copy(
                    src_ref=X_all.at[b], dst_ref=X_all.at[b],
                    send_sem=x_send.at[b], recv_sem=x_recv.at[b],
                    device_id=(my_x, b, my_z),
                    device_id_type=pl.DeviceIdType.MESH).wait_recv()
                pltpu.make_async_remote_copy(
                    src_ref=A_all.at[b], dst_ref=A_all.at[b],
                    send_sem=a_send.at[b], recv_sem=a_recv.at[b],
                    device_id=(my_x, b, my_z),
                    device_id_type=pl.DeviceIdType.MESH).wait_recv()

            xt = X_all[b]
            at = A_all[b]
            tot = None
            for le in range(E_LOCAL):
                e = my_y * E_LOCAL + le
                h = jnp.maximum(
                    jnp.dot(xt, w1_ref[le],
                            preferred_element_type=jnp.float32), 0.0)
                o = jnp.dot(h.astype(jnp.bfloat16), w2_ref[le],
                            preferred_element_type=jnp.float32)
                om = jnp.where(at == e, o, 0.0)
                tot = om if tot is None else tot + om
            acc[b] = tot.astype(jnp.bfloat16)

            if db == 0:
                orecv[my_y] = acc[my_y]
            else:
                r = pltpu.make_async_remote_copy(
                    src_ref=acc.at[b], dst_ref=orecv.at[my_y],
                    send_sem=o_send.at[b], recv_sem=o_recv.at[my_y],
                    device_id=(my_x, b, my_z),
                    device_id_type=pl.DeviceIdType.MESH)
                r.start()
                o_rdmas.append(r)

        for r in in_rdmas:
            r.wait_send()
        for dy in range(1, NY):
            s = (my_y + dy) % NY
            pltpu.make_async_remote_copy(
                src_ref=acc.at[s], dst_ref=orecv.at[s],
                send_sem=o_send.at[s], recv_sem=o_recv.at[s],
                device_id=(my_x, s, my_z),
                device_id_type=pl.DeviceIdType.MESH).wait_recv()
        for r in o_rdmas:
            r.wait_send()

        zbuf[my_z] = orecv[0] + orecv[1] + orecv[2] + orecv[3]

        z_rdmas = []
        for dz in range(1, NZ):
            p = (my_z + dz) % NZ
            r = pltpu.make_async_remote_copy(
                src_ref=zbuf.at[my_z], dst_ref=zbuf.at[my_z],
                send_sem=z_send.at[p], recv_sem=z_recv.at[my_z],
                device_id=(my_x, my_y, p),
                device_id_type=pl.DeviceIdType.MESH)
            r.start()
            z_rdmas.append(r)
        for dz in range(1, NZ):
            s = (my_z + dz) % NZ
            pltpu.make_async_remote_copy(
                src_ref=zbuf.at[s], dst_ref=zbuf.at[s],
                send_sem=z_send.at[s], recv_sem=z_recv.at[s],
                device_id=(my_x, my_y, s),
                device_id_type=pl.DeviceIdType.MESH).wait_recv()
        for r in z_rdmas:
            r.wait_send()

        for s in range(NZ):
            out_ref[pl.ds(s * QT, QT), :] = zbuf[s].astype(jnp.float32)

    return pl.pallas_call(
        body,
        out_shape=jax.ShapeDtypeStruct((T, D), jnp.float32),
        in_specs=[pl.BlockSpec(memory_space=pltpu.VMEM)] * 4,
        out_specs=pl.BlockSpec(memory_space=pltpu.VMEM),
        scratch_shapes=[
            pltpu.VMEM((NY, QT, D), jnp.bfloat16),
            pltpu.VMEM((NY, QT, 1), jnp.int32),
            pltpu.VMEM((NY, QT, D), jnp.bfloat16),
            pltpu.VMEM((NY, QT, D), jnp.bfloat16),
            pltpu.VMEM((NZ, QT, D), jnp.bfloat16),
            pltpu.SemaphoreType.DMA((NY,)),
            pltpu.SemaphoreType.DMA((NY,)),
            pltpu.SemaphoreType.DMA((NY,)),
            pltpu.SemaphoreType.DMA((NY,)),
            pltpu.SemaphoreType.DMA((NY,)),
            pltpu.SemaphoreType.DMA((NY,)),
            pltpu.SemaphoreType.DMA((NZ,)),
            pltpu.SemaphoreType.DMA((NZ,)),
        ],
        compiler_params=pltpu.CompilerParams(
            collective_id=0, vmem_limit_bytes=56 * 1024 * 1024),
    )(xb, a2, w1b, w2b)


# device time: 88288 ns/iter; 3.1374x vs baseline; 1.2934x over previous
import jax
import jax.numpy as jnp
from jax import lax
from jax.experimental import pallas as pl
from jax.experimental.pallas import tpu as pltpu

NY = 4
NZ = 4
E_LOCAL = 2
T = 1024
D = 1024
F = 2048
QT = T // NZ


def kernel(x, assign, W1, W2):
    a2 = assign.reshape(T, 1)

    def body(x_ref, a_ref, w1_ref, w2_ref, out_ref,
             X_all, A_all, acc, orecv, zbuf,
             xstage, w1stage, w2stage, w1b, w2b,
             x_send, x_recv, a_send, a_recv, o_send, o_recv,
             z_send, z_recv, local_sems):
        my_x = lax.axis_index("x")
        my_y = lax.axis_index("y")
        my_z = lax.axis_index("z")

        xcopy = pltpu.make_async_copy(
            x_ref.at[pl.ds(my_z * QT, QT), :], xstage, local_sems.at[0])
        xcopy.start()
        w1c = pltpu.make_async_copy(w1_ref.at[0], w1stage, local_sems.at[1])
        w1c.start()
        w2c = pltpu.make_async_copy(w2_ref.at[0], w2stage, local_sems.at[2])
        w2c.start()

        bsem = pltpu.get_barrier_semaphore()
        for dy in range(1, NY):
            pl.semaphore_signal(bsem, inc=1,
                                device_id=(my_x, (my_y + dy) % NY, my_z),
                                device_id_type=pl.DeviceIdType.MESH)
        for dz in range(1, NZ):
            pl.semaphore_signal(bsem, inc=1,
                                device_id=(my_x, my_y, (my_z + dz) % NZ),
                                device_id_type=pl.DeviceIdType.MESH)
        pl.semaphore_wait(bsem, (NY - 1) + (NZ - 1))

        xcopy.wait()
        X_all[my_y] = xstage[...].astype(jnp.bfloat16)
        A_all[my_y] = a_ref[pl.ds(my_z * QT, QT), :]

        in_rdmas = []
        for dy in range(1, NY):
            p = (my_y + dy) % NY
            r = pltpu.make_async_remote_copy(
                src_ref=X_all.at[my_y], dst_ref=X_all.at[my_y],
                send_sem=x_send.at[p], recv_sem=x_recv.at[my_y],
                device_id=(my_x, p, my_z),
                device_id_type=pl.DeviceIdType.MESH)
            r.start()
            in_rdmas.append(r)
            r = pltpu.make_async_remote_copy(
                src_ref=A_all.at[my_y], dst_ref=A_all.at[my_y],
                send_sem=a_send.at[p], recv_sem=a_recv.at[my_y],
                device_id=(my_x, p, my_z),
                device_id_type=pl.DeviceIdType.MESH)
            r.start()
            in_rdmas.append(r)

        w1c.wait()
        w1b[0] = w1stage[...].astype(jnp.bfloat16)
        w1c = pltpu.make_async_copy(w1_ref.at[1], w1stage, local_sems.at[1])
        w1c.start()
        w2c.wait()
        w2b[0] = w2stage[...].astype(jnp.bfloat16)
        w2c = pltpu.make_async_copy(w2_ref.at[1], w2stage, local_sems.at[2])
        w2c.start()
        w1c.wait()
        w1b[1] = w1stage[...].astype(jnp.bfloat16)
        w2c.wait()
        w2b[1] = w2stage[...].astype(jnp.bfloat16)

        o_rdmas = []
        for db in range(NY):
            b = (my_y + db) % NY
            if db > 0:
                pltpu.make_async_remote_copy(
                    src_ref=X_all.at[b], dst_ref=X_all.at[b],
                    send_sem=x_send.at[b], recv_sem=x_recv.at[b],
                    device_id=(my_x, b, my_z),
                    device_id_type=pl.DeviceIdType.MESH).wait_recv()
                pltpu.make_async_remote_copy(
                    src_ref=A_all.at[b], dst_ref=A_all.at[b],
                    send_sem=a_send.at[b], recv_sem=a_recv.at[b],
                    device_id=(my_x, b, my_z),
                    device_id_type=pl.DeviceIdType.MESH).wait_recv()

            xt = X_all[b]
            at = A_all[b]
            tot = None
            for le in range(E_LOCAL):
                e = my_y * E_LOCAL + le
                h = jnp.maximum(
                    jnp.dot(xt, w1b[le],
                            preferred_element_type=jnp.float32), 0.0)
                o = jnp.dot(h.astype(jnp.bfloat16), w2b[le],
                            preferred_element_type=jnp.float32)
                om = jnp.where(at == e, o, 0.0)
                tot = om if tot is None else tot + om
            acc[b] = tot.astype(jnp.bfloat16)

            if db == 0:
                orecv[my_y] = acc[my_y]
            else:
                r = pltpu.make_async_remote_copy(
                    src_ref=acc.at[b], dst_ref=orecv.at[my_y],
                    send_sem=o_send.at[b], recv_sem=o_recv.at[my_y],
                    device_id=(my_x, b, my_z),
                    device_id_type=pl.DeviceIdType.MESH)
                r.start()
                o_rdmas.append(r)

        for r in in_rdmas:
            r.wait_send()
        for dy in range(1, NY):
            s = (my_y + dy) % NY
            pltpu.make_async_remote_copy(
                src_ref=acc.at[s], dst_ref=orecv.at[s],
                send_sem=o_send.at[s], recv_sem=o_recv.at[s],
                device_id=(my_x, s, my_z),
                device_id_type=pl.DeviceIdType.MESH).wait_recv()
        for r in o_rdmas:
            r.wait_send()

        zbuf[my_z] = orecv[0] + orecv[1] + orecv[2] + orecv[3]

        z_rdmas = []
        for dz in range(1, NZ):
            p = (my_z + dz) % NZ
            r = pltpu.make_async_remote_copy(
                src_ref=zbuf.at[my_z], dst_ref=zbuf.at[my_z],
                send_sem=z_send.at[p], recv_sem=z_recv.at[my_z],
                device_id=(my_x, my_y, p),
                device_id_type=pl.DeviceIdType.MESH)
            r.start()
            z_rdmas.append(r)
        for dz in range(1, NZ):
            s = (my_z + dz) % NZ
            pltpu.make_async_remote_copy(
                src_ref=zbuf.at[s], dst_ref=zbuf.at[s],
                send_sem=z_send.at[s], recv_sem=z_recv.at[s],
                device_id=(my_x, my_y, s),
                device_id_type=pl.DeviceIdType.MESH).wait_recv()
        for r in z_rdmas:
            r.wait_send()

        for s in range(NZ):
            out_ref[pl.ds(s * QT, QT), :] = zbuf[s].astype(jnp.float32)

    return pl.pallas_call(
        body,
        out_shape=jax.ShapeDtypeStruct((T, D), jnp.float32),
        in_specs=[
            pl.BlockSpec(memory_space=pl.ANY),
            pl.BlockSpec(memory_space=pltpu.VMEM),
            pl.BlockSpec(memory_space=pl.ANY),
            pl.BlockSpec(memory_space=pl.ANY),
        ],
        out_specs=pl.BlockSpec(memory_space=pltpu.VMEM),
        scratch_shapes=[
            pltpu.VMEM((NY, QT, D), jnp.bfloat16),
            pltpu.VMEM((NY, QT, 1), jnp.int32),
            pltpu.VMEM((NY, QT, D), jnp.bfloat16),
            pltpu.VMEM((NY, QT, D), jnp.bfloat16),
            pltpu.VMEM((NZ, QT, D), jnp.bfloat16),
            pltpu.VMEM((QT, D), jnp.float32),
            pltpu.VMEM((D, F), jnp.float32),
            pltpu.VMEM((F, D), jnp.float32),
            pltpu.VMEM((E_LOCAL, D, F), jnp.bfloat16),
            pltpu.VMEM((E_LOCAL, F, D), jnp.bfloat16),
            pltpu.SemaphoreType.DMA((NY,)),
            pltpu.SemaphoreType.DMA((NY,)),
            pltpu.SemaphoreType.DMA((NY,)),
            pltpu.SemaphoreType.DMA((NY,)),
            pltpu.SemaphoreType.DMA((NY,)),
            pltpu.SemaphoreType.DMA((NY,)),
            pltpu.SemaphoreType.DMA((NZ,)),
            pltpu.SemaphoreType.DMA((NZ,)),
            pltpu.SemaphoreType.DMA((5,)),
        ],
        compiler_params=pltpu.CompilerParams(
            collective_id=0, vmem_limit_bytes=56 * 1024 * 1024),
    )(x, a2, W1, W2)


# device time: 69535 ns/iter; 3.9835x vs baseline; 1.2697x over previous
import jax
import jax.numpy as jnp
from jax import lax
from jax.experimental import pallas as pl
from jax.experimental.pallas import tpu as pltpu

NX = 2
NY = 4
NZ = 4
E_LOCAL = 2
T = 1024
D = 1024
F = 2048
QT = T // NZ
HR = QT // NX


def kernel(x, assign, W1, W2):
    a2 = assign.reshape(T, 1)

    def body(x_ref, a_ref, w1_ref, w2_ref, out_ref,
             X_all, A_all, acc, orecv, qbuf, zbuf,
             xstage, w1stage, w2stage, w1b, w2b,
             x_send, x_recv, a_send, a_recv, o_send, o_recv,
             qx_send, qx_recv, z_send, z_recv, local_sems):
        my_x = lax.axis_index("x")
        my_y = lax.axis_index("y")
        my_z = lax.axis_index("z")
        r0 = my_z * QT + my_x * HR

        xcopy = pltpu.make_async_copy(
            x_ref.at[pl.ds(r0, HR), :], xstage, local_sems.at[0])
        xcopy.start()
        w1c = pltpu.make_async_copy(w1_ref.at[0], w1stage, local_sems.at[1])
        w1c.start()
        w2c = pltpu.make_async_copy(w2_ref.at[0], w2stage, local_sems.at[2])
        w2c.start()

        bsem = pltpu.get_barrier_semaphore()
        for dy in range(1, NY):
            pl.semaphore_signal(bsem, inc=1,
                                device_id=(my_x, (my_y + dy) % NY, my_z),
                                device_id_type=pl.DeviceIdType.MESH)
        for dz in range(1, NZ):
            pl.semaphore_signal(bsem, inc=1,
                                device_id=(my_x, my_y, (my_z + dz) % NZ),
                                device_id_type=pl.DeviceIdType.MESH)
        pl.semaphore_signal(bsem, inc=1,
                            device_id=(1 - my_x, my_y, my_z),
                            device_id_type=pl.DeviceIdType.MESH)
        pl.semaphore_wait(bsem, (NY - 1) + (NZ - 1) + 1)

        xcopy.wait()
        X_all[my_y] = xstage[...].astype(jnp.bfloat16)
        A_all[my_y] = a_ref[pl.ds(r0, HR), :]

        in_rdmas = []
        for dy in range(1, NY):
            p = (my_y + dy) % NY
            r = pltpu.make_async_remote_copy(
                src_ref=X_all.at[my_y], dst_ref=X_all.at[my_y],
                send_sem=x_send.at[p], recv_sem=x_recv.at[my_y],
                device_id=(my_x, p, my_z),
                device_id_type=pl.DeviceIdType.MESH)
            r.start()
            in_rdmas.append(r)
            r = pltpu.make_async_remote_copy(
                src_ref=A_all.at[my_y], dst_ref=A_all.at[my_y],
                send_sem=a_send.at[p], recv_sem=a_recv.at[my_y],
                device_id=(my_x, p, my_z),
                device_id_type=pl.DeviceIdType.MESH)
            r.start()
            in_rdmas.append(r)

        w1c.wait()
        w1b[0] = w1stage[...].astype(jnp.bfloat16)
        w1c = pltpu.make_async_copy(w1_ref.at[1], w1stage, local_sems.at[1])
        w1c.start()
        w2c.wait()
        w2b[0] = w2stage[...].astype(jnp.bfloat16)
        w2c = pltpu.make_async_copy(w2_ref.at[1], w2stage, local_sems.at[2])
        w2c.start()
        w1c.wait()
        w1b[1] = w1stage[...].astype(jnp.bfloat16)
        w2c.wait()
        w2b[1] = w2stage[...].astype(jnp.bfloat16)

        o_rdmas = []
        for db in range(NY):
            b = (my_y + db) % NY
            if db > 0:
                pltpu.make_async_remote_copy(
                    src_ref=X_all.at[b], dst_ref=X_all.at[b],
                    send_sem=x_send.at[b], recv_sem=x_recv.at[b],
                    device_id=(my_x, b, my_z),
                    device_id_type=pl.DeviceIdType.MESH).wait_recv()
                pltpu.make_async_remote_copy(
                    src_ref=A_all.at[b], dst_ref=A_all.at[b],
                    send_sem=a_send.at[b], recv_sem=a_recv.at[b],
                    device_id=(my_x, b, my_z),
                    device_id_type=pl.DeviceIdType.MESH).wait_recv()

            xt = X_all[b]
            at = A_all[b]
            tot = None
            for le in range(E_LOCAL):
                e = my_y * E_LOCAL + le
                h = jnp.maximum(
                    jnp.dot(xt, w1b[le],
                            preferred_element_type=jnp.float32), 0.0)
                o = jnp.dot(h.astype(jnp.bfloat16), w2b[le],
                            preferred_element_type=jnp.float32)
                om = jnp.where(at == e, o, 0.0)
                tot = om if tot is None else tot + om
            acc[b] = tot.astype(jnp.bfloat16)

            if db == 0:
                orecv[my_y] = acc[my_y]
            else:
                r = pltpu.make_async_remote_copy(
                    src_ref=acc.at[b], dst_ref=orecv.at[my_y],
                    send_sem=o_send.at[b], recv_sem=o_recv.at[my_y],
                    device_id=(my_x, b, my_z),
                    device_id_type=pl.DeviceIdType.MESH)
                r.start()
                o_rdmas.append(r)

        for r in in_rdmas:
            r.wait_send()
        for dy in range(1, NY):
            s = (my_y + dy) % NY
            pltpu.make_async_remote_copy(
                src_ref=acc.at[s], dst_ref=orecv.at[s],
                send_sem=o_send.at[s], recv_sem=o_recv.at[s],
                device_id=(my_x, s, my_z),
                device_id_type=pl.DeviceIdType.MESH).wait_recv()
        for r in o_rdmas:
            r.wait_send()

        qbuf[my_x] = orecv[0] + orecv[1] + orecv[2] + orecv[3]

        qr = pltpu.make_async_remote_copy(
            src_ref=qbuf.at[my_x], dst_ref=qbuf.at[my_x],
            send_sem=qx_send.at[my_x], recv_sem=qx_recv.at[my_x],
            device_id=(1 - my_x, my_y, my_z),
            device_id_type=pl.DeviceIdType.MESH)
        qr.start()
        pltpu.make_async_remote_copy(
            src_ref=qbuf.at[1 - my_x], dst_ref=qbuf.at[1 - my_x],
            send_sem=qx_send.at[1 - my_x], recv_sem=qx_recv.at[1 - my_x],
            device_id=(1 - my_x, my_y, my_z),
            device_id_type=pl.DeviceIdType.MESH).wait_recv()
        qr.wait_send()

        zbuf[my_z] = qbuf[...]
        z_rdmas = []
        for dz in range(1, NZ):
            p = (my_z + dz) % NZ
            r = pltpu.make_async_remote_copy(
                src_ref=zbuf.at[my_z], dst_ref=zbuf.at[my_z],
                send_sem=z_send.at[p], recv_sem=z_recv.at[my_z],
                device_id=(my_x, my_y, p),
                device_id_type=pl.DeviceIdType.MESH)
            r.start()
            z_rdmas.append(r)
        for dz in range(1, NZ):
            s = (my_z + dz) % NZ
            pltpu.make_async_remote_copy(
                src_ref=zbuf.at[s], dst_ref=zbuf.at[s],
                send_sem=z_send.at[s], recv_sem=z_recv.at[s],
                device_id=(my_x, my_y, s),
                device_id_type=pl.DeviceIdType.MESH).wait_recv()
        for r in z_rdmas:
            r.wait_send()

        for s in range(NZ):
            for hx in range(NX):
                out_ref[pl.ds(s * QT + hx * HR, HR), :] = (
                    zbuf[s, hx].astype(jnp.float32))

    return pl.pallas_call(
        body,
        out_shape=jax.ShapeDtypeStruct((T, D), jnp.float32),
        in_specs=[
            pl.BlockSpec(memory_space=pl.ANY),
            pl.BlockSpec(memory_space=pltpu.VMEM),
            pl.BlockSpec(memory_space=pl.ANY),
            pl.BlockSpec(memory_space=pl.ANY),
        ],
        out_specs=pl.BlockSpec(memory_space=pltpu.VMEM),
        scratch_shapes=[
            pltpu.VMEM((NY, HR, D), jnp.bfloat16),
            pltpu.VMEM((NY, HR, 1), jnp.int32),
            pltpu.VMEM((NY, HR, D), jnp.bfloat16),
            pltpu.VMEM((NY, HR, D), jnp.bfloat16),
            pltpu.VMEM((NX, HR, D), jnp.bfloat16),
            pltpu.VMEM((NZ, NX, HR, D), jnp.bfloat16),
            pltpu.VMEM((HR, D), jnp.float32),
            pltpu.VMEM((D, F), jnp.float32),
            pltpu.VMEM((F, D), jnp.float32),
            pltpu.VMEM((E_LOCAL, D, F), jnp.bfloat16),
            pltpu.VMEM((E_LOCAL, F, D), jnp.bfloat16),
            pltpu.SemaphoreType.DMA((NY,)),
            pltpu.SemaphoreType.DMA((NY,)),
            pltpu.SemaphoreType.DMA((NY,)),
            pltpu.SemaphoreType.DMA((NY,)),
            pltpu.SemaphoreType.DMA((NY,)),
            pltpu.SemaphoreType.DMA((NY,)),
            pltpu.SemaphoreType.DMA((NX,)),
            pltpu.SemaphoreType.DMA((NX,)),
            pltpu.SemaphoreType.DMA((NZ,)),
            pltpu.SemaphoreType.DMA((NZ,)),
            pltpu.SemaphoreType.DMA((3,)),
        ],
        compiler_params=pltpu.CompilerParams(
            collective_id=0, vmem_limit_bytes=56 * 1024 * 1024),
    )(x, a2, W1, W2)


# device time: 66831 ns/iter; 4.1447x vs baseline; 1.0405x over previous
import jax
import jax.numpy as jnp
from jax import lax
from jax.experimental import pallas as pl
from jax.experimental.pallas import tpu as pltpu

NX = 2
NY = 4
NZ = 4
E_LOCAL = 2
T = 1024
D = 1024
F = 2048
QT = T // NZ
HR = QT // NX


def kernel(x, assign, W1, W2):
    a2 = assign.reshape(T, 1)

    def body(x_ref, a_ref, w1_ref, w2_ref, out_ref,
             X_all, A_all, acc, orecv, qbuf, zbuf,
             xstage, w1stage, w2stage, w1b, w2b,
             x_send, x_recv, a_send, a_recv, o_send, o_recv,
             qx_send, qx_recv, z_send, z_recv, local_sems):
        my_x = lax.axis_index("x")
        my_y = lax.axis_index("y")
        my_z = lax.axis_index("z")
        r0 = my_z * QT + my_x * HR

        xcopy = pltpu.make_async_copy(
            x_ref.at[pl.ds(r0, HR), :], xstage, local_sems.at[0])
        xcopy.start()
        w1c = pltpu.make_async_copy(w1_ref.at[0], w1stage, local_sems.at[1])
        w1c.start()
        w2c = pltpu.make_async_copy(w2_ref.at[0], w2stage, local_sems.at[2])
        w2c.start()

        bsem = pltpu.get_barrier_semaphore()
        for dy in range(1, NY):
            pl.semaphore_signal(bsem, inc=1,
                                device_id=(my_x, (my_y + dy) % NY, my_z),
                                device_id_type=pl.DeviceIdType.MESH)
        for dz in range(1, NZ):
            pl.semaphore_signal(bsem, inc=1,
                                device_id=(my_x, my_y, (my_z + dz) % NZ),
                                device_id_type=pl.DeviceIdType.MESH)
        pl.semaphore_signal(bsem, inc=1,
                            device_id=(1 - my_x, my_y, my_z),
                            device_id_type=pl.DeviceIdType.MESH)
        pl.semaphore_wait(bsem, (NY - 1) + (NZ - 1) + 1)

        xcopy.wait()
        X_all[my_y] = xstage[...].astype(jnp.bfloat16)
        A_all[my_y] = a_ref[pl.ds(r0, HR), :]

        in_rdmas = []
        for dy in range(1, NY):
            p = (my_y + dy) % NY
            r = pltpu.make_async_remote_copy(
                src_ref=X_all.at[my_y], dst_ref=X_all.at[my_y],
                send_sem=x_send.at[p], recv_sem=x_recv.at[my_y],
                device_id=(my_x, p, my_z),
                device_id_type=pl.DeviceIdType.MESH)
            r.start()
            in_rdmas.append(r)
            r = pltpu.make_async_remote_copy(
                src_ref=A_all.at[my_y], dst_ref=A_all.at[my_y],
                send_sem=a_send.at[p], recv_sem=a_recv.at[my_y],
                device_id=(my_x, p, my_z),
                device_id_type=pl.DeviceIdType.MESH)
            r.start()
            in_rdmas.append(r)

        w1c.wait()
        w1b[0] = w1stage[...].astype(jnp.bfloat16)
        w1c = pltpu.make_async_copy(w1_ref.at[1], w1stage, local_sems.at[1])
        w1c.start()
        w2c.wait()
        w2b[0] = w2stage[...].astype(jnp.bfloat16)
        w2c = pltpu.make_async_copy(w2_ref.at[1], w2stage, local_sems.at[2])
        w2c.start()
        w1c.wait()
        w1b[1] = w1stage[...].astype(jnp.bfloat16)
        w2c.wait()
        w2b[1] = w2stage[...].astype(jnp.bfloat16)

        o_rdmas = []
        for db in range(NY):
            b = (my_y + db) % NY
            if db > 0:
                pltpu.make_async_remote_copy(
                    src_ref=X_all.at[b], dst_ref=X_all.at[b],
                    send_sem=x_send.at[b], recv_sem=x_recv.at[b],
                    device_id=(my_x, b, my_z),
                    device_id_type=pl.DeviceIdType.MESH).wait_recv()
                pltpu.make_async_remote_copy(
                    src_ref=A_all.at[b], dst_ref=A_all.at[b],
                    send_sem=a_send.at[b], recv_sem=a_recv.at[b],
                    device_id=(my_x, b, my_z),
                    device_id_type=pl.DeviceIdType.MESH).wait_recv()

            xt = X_all[b]
            at = A_all[b]
            tot = None
            for le in range(E_LOCAL):
                e = my_y * E_LOCAL + le
                h = jnp.maximum(
                    jnp.dot(xt, w1b[le],
                            preferred_element_type=jnp.float32), 0.0)
                o = jnp.dot(h.astype(jnp.bfloat16), w2b[le],
                            preferred_element_type=jnp.float32)
                om = jnp.where(at == e, o, 0.0)
                tot = om if tot is None else tot + om
            acc[b] = tot.astype(jnp.bfloat16)

            if db == 0:
                orecv[my_y] = acc[my_y]
            else:
                r = pltpu.make_async_remote_copy(
                    src_ref=acc.at[b], dst_ref=orecv.at[my_y],
                    send_sem=o_send.at[b], recv_sem=o_recv.at[my_y],
                    device_id=(my_x, b, my_z),
                    device_id_type=pl.DeviceIdType.MESH)
                r.start()
                o_rdmas.append(r)

        for r in in_rdmas:
            r.wait_send()
        for dy in range(1, NY):
            s = (my_y + dy) % NY
            pltpu.make_async_remote_copy(
                src_ref=acc.at[s], dst_ref=orecv.at[s],
                send_sem=o_send.at[s], recv_sem=o_recv.at[s],
                device_id=(my_x, s, my_z),
                device_id_type=pl.DeviceIdType.MESH).wait_recv()
        for r in o_rdmas:
            r.wait_send()

        qbuf[my_x] = orecv[0] + orecv[1] + orecv[2] + orecv[3]
        zbuf[my_z, my_x] = qbuf[my_x]

        z_rdmas = []
        for dz in range(1, NZ):
            p = (my_z + dz) % NZ
            r = pltpu.make_async_remote_copy(
                src_ref=zbuf.at[my_z, my_x], dst_ref=zbuf.at[my_z, my_x],
                send_sem=z_send.at[p, 0], recv_sem=z_recv.at[my_z, 0],
                device_id=(my_x, my_y, p),
                device_id_type=pl.DeviceIdType.MESH)
            r.start()
            z_rdmas.append(r)
        qr = pltpu.make_async_remote_copy(
            src_ref=qbuf.at[my_x], dst_ref=qbuf.at[my_x],
            send_sem=qx_send.at[my_x], recv_sem=qx_recv.at[my_x],
            device_id=(1 - my_x, my_y, my_z),
            device_id_type=pl.DeviceIdType.MESH)
        qr.start()

        out_ref[pl.ds(my_z * QT + my_x * HR, HR), :] = (
            qbuf[my_x].astype(jnp.float32))

        pltpu.make_async_remote_copy(
            src_ref=qbuf.at[1 - my_x], dst_ref=qbuf.at[1 - my_x],
            send_sem=qx_send.at[1 - my_x], recv_sem=qx_recv.at[1 - my_x],
            device_id=(1 - my_x, my_y, my_z),
            device_id_type=pl.DeviceIdType.MESH).wait_recv()
        qr.wait_send()
        zbuf[my_z, 1 - my_x] = qbuf[1 - my_x]
        for dz in range(1, NZ):
            p = (my_z + dz) % NZ
            r = pltpu.make_async_remote_copy(
                src_ref=zbuf.at[my_z, 1 - my_x],
                dst_ref=zbuf.at[my_z, 1 - my_x],
                send_sem=z_send.at[p, 1], recv_sem=z_recv.at[my_z, 1],
                device_id=(my_x, my_y, p),
                device_id_type=pl.DeviceIdType.MESH)
            r.start()
            z_rdmas.append(r)

        out_ref[pl.ds(my_z * QT + (1 - my_x) * HR, HR), :] = (
            qbuf[1 - my_x].astype(jnp.float32))

        for h, hx in ((0, my_x), (1, 1 - my_x)):
            for dz in range(1, NZ):
                s = (my_z + dz) % NZ
                pltpu.make_async_remote_copy(
                    src_ref=zbuf.at[s, hx], dst_ref=zbuf.at[s, hx],
                    send_sem=z_send.at[s, h], recv_sem=z_recv.at[s, h],
                    device_id=(my_x, my_y, s),
                    device_id_type=pl.DeviceIdType.MESH).wait_recv()
                out_ref[pl.ds(s * QT + hx * HR, HR), :] = (
                    zbuf[s, hx].astype(jnp.float32))
        for r in z_rdmas:
            r.wait_send()

    return pl.pallas_call(
        body,
        out_shape=jax.ShapeDtypeStruct((T, D), jnp.float32),
        in_specs=[
            pl.BlockSpec(memory_space=pl.ANY),
            pl.BlockSpec(memory_space=pltpu.VMEM),
            pl.BlockSpec(memory_space=pl.ANY),
            pl.BlockSpec(memory_space=pl.ANY),
        ],
        out_specs=pl.BlockSpec(memory_space=pltpu.VMEM),
        scratch_shapes=[
            pltpu.VMEM((NY, HR, D), jnp.bfloat16),
            pltpu.VMEM((NY, HR, 1), jnp.int32),
            pltpu.VMEM((NY, HR, D), jnp.bfloat16),
            pltpu.VMEM((NY, HR, D), jnp.bfloat16),
            pltpu.VMEM((NX, HR, D), jnp.bfloat16),
            pltpu.VMEM((NZ, NX, HR, D), jnp.bfloat16),
            pltpu.VMEM((HR, D), jnp.float32),
            pltpu.VMEM((D, F), jnp.float32),
            pltpu.VMEM((F, D), jnp.float32),
            pltpu.VMEM((E_LOCAL, D, F), jnp.bfloat16),
            pltpu.VMEM((E_LOCAL, F, D), jnp.bfloat16),
            pltpu.SemaphoreType.DMA((NY,)),
            pltpu.SemaphoreType.DMA((NY,)),
            pltpu.SemaphoreType.DMA((NY,)),
            pltpu.SemaphoreType.DMA((NY,)),
            pltpu.SemaphoreType.DMA((NY,)),
            pltpu.SemaphoreType.DMA((NY,)),
            pltpu.SemaphoreType.DMA((NX,)),
            pltpu.SemaphoreType.DMA((NX,)),
            pltpu.SemaphoreType.DMA((NZ, 2)),
            pltpu.SemaphoreType.DMA((NZ, 2)),
            pltpu.SemaphoreType.DMA((3,)),
        ],
        compiler_params=pltpu.CompilerParams(
            collective_id=0, vmem_limit_bytes=56 * 1024 * 1024),
    )(x, a2, W1, W2)
